# baseline (speedup 1.0000x reference)
import functools

import jax
import jax.numpy as jnp
import numpy as np
from jax import lax
from jax.experimental import pallas as pl
from jax.experimental.pallas import tpu as pltpu
from jax.experimental.pallas import tpu_sc as plsc

B = 32
N = 2048
D = 3
L = 16
NV = N // L

THETA = float(np.float32(-1.0 / np.sqrt(3.0)))


def _vsort(a):
    sk, _ = plsc.sort_key_val(a, a)
    return sk


def _apply_group(regs, stages):
    G = len(regs)
    for st in stages:
        if st == "s":
            for i in range(G):
                regs[i] = _vsort(regs[i])
        elif st[0] == "R":
            sub = st[1]
            for q in range(0, G, sub):
                for i in range(sub // 2):
                    a = regs[q + i]
                    b = lax.rev(regs[q + sub - 1 - i], (0,))
                    regs[q + i] = jnp.minimum(a, b)
                    regs[q + sub - 1 - i] = lax.rev(jnp.maximum(a, b), (0,))
        else:
            dv = st[1]
            for i in range(G):
                if (i & dv) == 0:
                    a, b = regs[i], regs[i + dv]
                    regs[i] = jnp.minimum(a, b)
                    regs[i + dv] = jnp.maximum(a, b)
    return regs


def _fused_pass(bufs, G, stages, unroll=2):
    @plsc.parallel_loop(0, NV // G, unroll=unroll)
    def body(g):
        g0 = g * G
        for buf in bufs:
            regs = [buf[pl.ds((g0 + i) * L, L)] for i in range(G)]
            regs = _apply_group(regs, stages)
            for i in range(G):
                buf[pl.ds((g0 + i) * L, L)] = regs[i]


def _rev_pass(bufs, mv2):
    hv = mv2 // 2
    sh = hv.bit_length() - 1

    @plsc.parallel_loop(0, NV // 2, unroll=8)
    def body(j):
        r = j >> sh
        jj = j & (hv - 1)
        vi = r * mv2 + jj
        wi = r * mv2 + (mv2 - 1) - jj
        for buf in bufs:
            a = buf[pl.ds(vi * L, L)]
            b = lax.rev(buf[pl.ds(wi * L, L)], (0,))
            buf[pl.ds(vi * L, L)] = jnp.minimum(a, b)
            buf[pl.ds(wi * L, L)] = lax.rev(jnp.maximum(a, b), (0,))


def _d_pass(bufs, dv):
    sd = dv.bit_length() - 1

    @plsc.parallel_loop(0, NV // 2, unroll=8)
    def body(j):
        vi = ((j >> sd) << (sd + 1)) | (j & (dv - 1))
        for buf in bufs:
            a = buf[pl.ds(vi * L, L)]
            b = buf[pl.ds((vi + dv) * L, L)]
            buf[pl.ds(vi * L, L)] = jnp.minimum(a, b)
            buf[pl.ds((vi + dv) * L, L)] = jnp.maximum(a, b)


_P1 = ["s", ("R", 2), "s", ("R", 4), ("d", 1), "s",
       ("R", 8), ("d", 2), ("d", 1), "s"]
_T16R = [("R", 16), ("d", 4), ("d", 2), ("d", 1), "s"]
_T16 = [("d", 8), ("d", 4), ("d", 2), ("d", 1), "s"]

_SC_MESH = plsc.VectorSubcoreMesh(core_axis_name="c", subcore_axis_name="s")


@functools.partial(
    pl.kernel,
    out_type=jax.ShapeDtypeStruct((B, L), jnp.float32),
    mesh=_SC_MESH,
    compiler_params=pltpu.CompilerParams(needs_layout_passes=False),
    scratch_types=[
        pltpu.VMEM((N * D,), jnp.float32),
        pltpu.VMEM((N * D,), jnp.float32),
        pltpu.VMEM((N,), jnp.float32),
        pltpu.VMEM((N,), jnp.float32),
        pltpu.VMEM((L,), jnp.float32),
        pltpu.SemaphoreType.DMA,
        pltpu.SemaphoreType.DMA,
    ],
)
def _swd_sc(x_hbm, y_hbm, out_hbm, xraw, yraw, u, v, ovec, sem_x, sem_y):
    b = lax.axis_index("s") * 2 + lax.axis_index("c")

    cx = pltpu.async_copy(x_hbm.at[b], xraw, sem_x)
    cy = pltpu.async_copy(y_hbm.at[b], yraw, sem_y)
    cx.wait()
    cy.wait()

    @plsc.parallel_loop(0, NV // 8)
    def proj_body(g):
        g0 = g * 8
        for raw, out in ((xraw, u), (yraw, v)):
            regs = []
            for i in range(8):
                j = g0 + i
                p0 = raw[pl.ds(j * L, L)]
                p1 = raw[pl.ds(N + j * L, L)]
                p2 = raw[pl.ds(2 * N + j * L, L)]
                regs.append((p0 + p1 + p2) * THETA)
            regs = _apply_group(regs, _P1)
            for i in range(8):
                out[pl.ds((g0 + i) * L, L)] = regs[i]

    bufs = (u, v)
    _fused_pass(bufs, 16, _T16R)
    _rev_pass(bufs, 32)
    _fused_pass(bufs, 16, _T16)
    _rev_pass(bufs, 64)
    _d_pass(bufs, 16)
    _fused_pass(bufs, 16, _T16)

    _rev_pass(bufs, 128)
    for dv in (32, 16):
        _d_pass(bufs, dv)

    zero = jnp.zeros((L,), jnp.float32)

    @plsc.parallel_loop(0, NV // 16, carry=(zero, zero))
    def diff_acc(g, acc):
        g0 = g * 16
        ru = [u[pl.ds((g0 + i) * L, L)] for i in range(16)]
        ru = _apply_group(ru, _T16)
        rv = [v[pl.ds((g0 + i) * L, L)] for i in range(16)]
        rv = _apply_group(rv, _T16)
        a0, a1 = acc
        for i in range(16):
            d = ru[i] - rv[i]
            if i % 2 == 0:
                a0 = a0 + d * d
            else:
                a1 = a1 + d * d
        return (a0, a1)

    acc0, acc1 = diff_acc
    s = jnp.sum(acc0 + acc1)
    ovec[...] = jnp.broadcast_to(s, (L,))
    pltpu.sync_copy(ovec, out_hbm.at[b])


def _finish_tc(s_ref, o_ref):
    o_ref[0, 0] = jnp.sum(jnp.sqrt(s_ref[...])) * jnp.float32(1.0 / (B * L))


_finish = pl.pallas_call(
    _finish_tc,
    out_shape=jax.ShapeDtypeStruct((1, 1), jnp.float32),
    out_specs=pl.BlockSpec(memory_space=pltpu.SMEM),
)


def kernel(x, y):
    xf = x.transpose(0, 2, 1).reshape(B, D * N)
    yf = y.transpose(0, 2, 1).reshape(B, D * N)
    s = _swd_sc(xf, yf)
    return _finish(s)[0, 0]


# speedup vs baseline: 1.0264x; 1.0264x over previous
import functools

import jax
import jax.numpy as jnp
import numpy as np
from jax import lax
from jax.experimental import pallas as pl
from jax.experimental.pallas import tpu as pltpu
from jax.experimental.pallas import tpu_sc as plsc

B = 32
N = 2048
D = 3
L = 16
NV = N // L

THETA = float(np.float32(-1.0 / np.sqrt(3.0)))


def _vsort(a):
    sk, _ = plsc.sort_key_val(a, a)
    return sk


def _apply_group(regs, stages):
    G = len(regs)
    for st in stages:
        if st == "s":
            for i in range(G):
                regs[i] = _vsort(regs[i])
        elif st[0] == "R":
            sub = st[1]
            for q in range(0, G, sub):
                for i in range(sub // 2):
                    a = regs[q + i]
                    b = lax.rev(regs[q + sub - 1 - i], (0,))
                    regs[q + i] = jnp.minimum(a, b)
                    regs[q + sub - 1 - i] = lax.rev(jnp.maximum(a, b), (0,))
        else:
            dv = st[1]
            for i in range(G):
                if (i & dv) == 0:
                    a, b = regs[i], regs[i + dv]
                    regs[i] = jnp.minimum(a, b)
                    regs[i + dv] = jnp.maximum(a, b)
    return regs


def _fused_pass(bufs, G, stages, unroll=1):
    @plsc.parallel_loop(0, NV // G, unroll=unroll)
    def body(g):
        g0 = g * G
        for buf in bufs:
            regs = [buf[pl.ds((g0 + i) * L, L)] for i in range(G)]
            regs = _apply_group(regs, stages)
            for i in range(G):
                buf[pl.ds((g0 + i) * L, L)] = regs[i]


def _rev_pass(bufs, mv2):
    hv = mv2 // 2
    sh = hv.bit_length() - 1

    @plsc.parallel_loop(0, NV // 2, unroll=4)
    def body(j):
        r = j >> sh
        jj = j & (hv - 1)
        vi = r * mv2 + jj
        wi = r * mv2 + (mv2 - 1) - jj
        for buf in bufs:
            a = buf[pl.ds(vi * L, L)]
            b = lax.rev(buf[pl.ds(wi * L, L)], (0,))
            buf[pl.ds(vi * L, L)] = jnp.minimum(a, b)
            buf[pl.ds(wi * L, L)] = lax.rev(jnp.maximum(a, b), (0,))


def _d_pass(bufs, dv):
    sd = dv.bit_length() - 1

    @plsc.parallel_loop(0, NV // 2, unroll=4)
    def body(j):
        vi = ((j >> sd) << (sd + 1)) | (j & (dv - 1))
        for buf in bufs:
            a = buf[pl.ds(vi * L, L)]
            b = buf[pl.ds((vi + dv) * L, L)]
            buf[pl.ds(vi * L, L)] = jnp.minimum(a, b)
            buf[pl.ds((vi + dv) * L, L)] = jnp.maximum(a, b)


_P1 = ["s", ("R", 2), "s", ("R", 4), ("d", 1), "s",
       ("R", 8), ("d", 2), ("d", 1), "s"]
_T16R = [("R", 16), ("d", 4), ("d", 2), ("d", 1), "s"]
_T16 = [("d", 8), ("d", 4), ("d", 2), ("d", 1), "s"]

_SC_MESH = plsc.VectorSubcoreMesh(core_axis_name="c", subcore_axis_name="s")


@functools.partial(
    pl.kernel,
    out_type=jax.ShapeDtypeStruct((B, L), jnp.float32),
    mesh=_SC_MESH,
    compiler_params=pltpu.CompilerParams(needs_layout_passes=False),
    scratch_types=[
        pltpu.VMEM((N * D,), jnp.float32),
        pltpu.VMEM((N * D,), jnp.float32),
        pltpu.VMEM((N,), jnp.float32),
        pltpu.VMEM((N,), jnp.float32),
        pltpu.VMEM((L,), jnp.float32),
        pltpu.SemaphoreType.DMA,
        pltpu.SemaphoreType.DMA,
    ],
)
def _swd_sc(x_hbm, y_hbm, out_hbm, xraw, yraw, u, v, ovec, sem_x, sem_y):
    b = lax.axis_index("s") * 2 + lax.axis_index("c")

    cx = pltpu.async_copy(x_hbm.at[b], xraw, sem_x)
    cy = pltpu.async_copy(y_hbm.at[b], yraw, sem_y)
    cx.wait()
    cy.wait()

    @plsc.parallel_loop(0, NV // 8)
    def proj_body(g):
        g0 = g * 8
        for raw, out in ((xraw, u), (yraw, v)):
            regs = []
            for i in range(8):
                j = g0 + i
                p0 = raw[pl.ds(j * L, L)]
                p1 = raw[pl.ds(N + j * L, L)]
                p2 = raw[pl.ds(2 * N + j * L, L)]
                regs.append((p0 + p1 + p2) * THETA)
            regs = _apply_group(regs, _P1)
            for i in range(8):
                out[pl.ds((g0 + i) * L, L)] = regs[i]

    bufs = (u, v)
    _fused_pass(bufs, 16, _T16R)
    _rev_pass(bufs, 32)
    _fused_pass(bufs, 16, _T16)
    _rev_pass(bufs, 64)
    _d_pass(bufs, 16)
    _fused_pass(bufs, 16, _T16)

    _rev_pass(bufs, 128)
    for dv in (32, 16):
        _d_pass(bufs, dv)

    zero = jnp.zeros((L,), jnp.float32)

    @plsc.parallel_loop(0, NV // 16, carry=(zero, zero))
    def diff_acc(g, acc):
        g0 = g * 16
        ru = [u[pl.ds((g0 + i) * L, L)] for i in range(16)]
        ru = _apply_group(ru, _T16)
        rv = [v[pl.ds((g0 + i) * L, L)] for i in range(16)]
        rv = _apply_group(rv, _T16)
        a0, a1 = acc
        for i in range(16):
            d = ru[i] - rv[i]
            if i % 2 == 0:
                a0 = a0 + d * d
            else:
                a1 = a1 + d * d
        return (a0, a1)

    acc0, acc1 = diff_acc
    s = jnp.sum(acc0 + acc1)
    ovec[...] = jnp.broadcast_to(s, (L,))
    pltpu.sync_copy(ovec, out_hbm.at[b])


def _finish_tc(s_ref, o_ref):
    o_ref[0, 0] = jnp.sum(jnp.sqrt(s_ref[...])) * jnp.float32(1.0 / (B * L))


_finish = pl.pallas_call(
    _finish_tc,
    out_shape=jax.ShapeDtypeStruct((1, 1), jnp.float32),
    out_specs=pl.BlockSpec(memory_space=pltpu.SMEM),
)


def kernel(x, y):
    xf = x.transpose(0, 2, 1).reshape(B, D * N)
    yf = y.transpose(0, 2, 1).reshape(B, D * N)
    s = _swd_sc(xf, yf)
    return _finish(s)[0, 0]
